# baseline (speedup 1.0000x reference)
import jax
import jax.numpy as jnp
from jax import lax
from jax.experimental import pallas as pl
from jax.experimental.pallas import tpu as pltpu
from jax.experimental.pallas import tpu_sc as plsc

_NCORES = 2
_NSUB = 16
_NWORKERS = _NCORES * _NSUB
_NSLOTS = 6
_LOOKAHEAD = 4
_K = 4
_TC_GROUP = 16


_GROUP = 4
_RING = 4


def _sc_gather_chunk(k, chunk_batches, seq, dim, table, idx32):
    per_worker = chunk_batches // _NWORKERS
    ngroups = per_worker // _GROUP
    mesh = plsc.VectorSubcoreMesh(core_axis_name="c", subcore_axis_name="s")
    sem_types = [pltpu.SemaphoreType.DMA] * (2 * _RING)

    @pl.kernel(
        out_type=jax.ShapeDtypeStruct((chunk_batches * seq, dim), table.dtype),
        mesh=mesh,
        scratch_types=[
            pltpu.VMEM((per_worker, seq), jnp.int32),
            pltpu.VMEM((_RING, _GROUP * seq, dim), table.dtype),
        ] + sem_types,
    )
    def gather_kernel(table_hbm, idx_hbm, out_hbm, idx_v, rows_v, *sems):
        g_sems = sems[:_RING]
        s_sems = sems[_RING:]
        wid = lax.axis_index("s") * _NCORES + lax.axis_index("c")
        base = wid * per_worker

        pltpu.sync_copy(
            idx_hbm.at[pl.ds(k * chunk_batches + base, per_worker)], idx_v)

        def fire(c, slot):
            for j in range(_GROUP):
                pltpu.async_copy(
                    table_hbm.at[idx_v.at[c * _GROUP + j]],
                    rows_v.at[slot, pl.ds(j * seq, seq)], g_sems[slot])

        def wait_gather(slot):
            for j in range(_GROUP):
                pltpu.make_async_copy(
                    table_hbm.at[idx_v.at[0]],
                    rows_v.at[slot, pl.ds(j * seq, seq)], g_sems[slot]).wait()

        def store(c, slot):
            pltpu.async_copy(
                rows_v.at[slot],
                out_hbm.at[pl.ds((base + c * _GROUP) * seq, _GROUP * seq)],
                s_sems[slot])

        def wait_store(slot):
            pltpu.make_async_copy(
                rows_v.at[slot],
                out_hbm.at[pl.ds(0, _GROUP * seq)], s_sems[slot]).wait()

        fire(0, 0)
        fire(1, 1)
        for c in range(ngroups):
            slot = c % _RING
            wait_gather(slot)
            store(c, slot)
            nxt = c + 2
            if nxt < ngroups:
                nxt_slot = nxt % _RING
                if nxt >= _RING:
                    wait_store(nxt_slot)
                fire(nxt, nxt_slot)
        for s in range(_RING):
            wait_store(s)

    return gather_kernel(table, idx32)


def _tc_relayout_chunk(k, chunk_batches, seq, dim, flat, carry, batch, first):
    grid = chunk_batches // _TC_GROUP
    blocks_per_chunk = grid

    def body(*refs):
        if first:
            in_ref, o_ref = refs
        else:
            in_ref, _, o_ref = refs
        for j in range(_TC_GROUP):
            o_ref[j] = in_ref[pl.ds(j * seq, seq), :]

    in_specs = [pl.BlockSpec((_TC_GROUP * seq, dim), lambda i: (i, 0))]
    operands = (flat,)
    kwargs = {}
    if not first:
        in_specs.append(pl.BlockSpec(memory_space=pl.ANY))
        operands = (flat, carry)
        kwargs["input_output_aliases"] = {1: 0}

    return pl.pallas_call(
        body,
        grid=(grid,),
        in_specs=in_specs,
        out_specs=pl.BlockSpec(
            (_TC_GROUP, seq, dim),
            lambda i, _k=k, _b=blocks_per_chunk: (_k * _b + i, 0, 0)),
        out_shape=jax.ShapeDtypeStruct((batch, seq, dim), flat.dtype),
        **kwargs,
    )(*operands)


def kernel(indices, table):
    batch, seq = indices.shape
    num_rows, dim = table.shape
    idx32 = indices.astype(jnp.int32)
    chunk_batches = batch // _K

    out = None
    for k in range(_K):
        flat = _sc_gather_chunk(k, chunk_batches, seq, dim, table, idx32)
        out = _tc_relayout_chunk(k, chunk_batches, seq, dim, flat, out,
                                 batch, first=(k == 0))
    return out


# speedup vs baseline: 1.1030x; 1.1030x over previous
import jax
import jax.numpy as jnp
from jax import lax
from jax.experimental import pallas as pl
from jax.experimental.pallas import tpu as pltpu
from jax.experimental.pallas import tpu_sc as plsc

_NCORES = 2
_NSUB = 16
_NWORKERS = _NCORES * _NSUB
_NSLOTS = 6
_LOOKAHEAD = 4
_K = 4


def _sc_gather_chunk(k, chunk_batches, seq, dim, table, idx32):
    per_worker = chunk_batches // _NWORKERS
    mesh = plsc.VectorSubcoreMesh(core_axis_name="c", subcore_axis_name="s")
    sem_types = [pltpu.SemaphoreType.DMA] * (2 * _NSLOTS)

    @pl.kernel(
        out_type=jax.ShapeDtypeStruct((chunk_batches, seq, dim), table.dtype),
        mesh=mesh,
        scratch_types=[
            pltpu.VMEM((per_worker, seq), jnp.int32),
            pltpu.VMEM((_NSLOTS, seq, dim), table.dtype),
        ] + sem_types,
    )
    def gather_kernel(table_hbm, idx_hbm, out_hbm, idx_v, rows_v, *sems):
        g_sems = sems[:_NSLOTS]
        s_sems = sems[_NSLOTS:]
        wid = lax.axis_index("s") * _NCORES + lax.axis_index("c")
        base = wid * per_worker

        pltpu.sync_copy(
            idx_hbm.at[pl.ds(k * chunk_batches + base, per_worker)], idx_v)

        def fire(c, slot):
            pltpu.async_copy(
                table_hbm.at[idx_v.at[c]], rows_v.at[slot], g_sems[slot])

        def wait_gather(slot):
            pltpu.make_async_copy(
                table_hbm.at[idx_v.at[0]],
                rows_v.at[slot], g_sems[slot]).wait()

        def store(c, slot):
            pltpu.async_copy(
                rows_v.at[slot], out_hbm.at[base + c], s_sems[slot])

        def wait_store(slot):
            pltpu.make_async_copy(
                rows_v.at[slot], out_hbm.at[base], s_sems[slot]).wait()

        for c in range(_LOOKAHEAD):
            fire(c, c % _NSLOTS)

        def chunk_body(c, slot, fire_next, wait_prev_store):
            wait_gather(slot)
            store(c, slot)
            if fire_next:
                nxt_slot = (slot + _LOOKAHEAD) % _NSLOTS
                if wait_prev_store:
                    wait_store(nxt_slot)
                fire(c + _LOOKAHEAD, nxt_slot)

        chunk_body(0, 0, True, False)
        chunk_body(1, 1, True, False)

        steady = ((per_worker - _LOOKAHEAD - 2) // _NSLOTS) * _NSLOTS

        @pl.loop(2, 2 + steady, step=_NSLOTS)
        def _(c0):
            for j in range(_NSLOTS):
                chunk_body(c0 + j, (2 + j) % _NSLOTS, True, True)

        c = 2 + steady
        while c + _LOOKAHEAD < per_worker:
            chunk_body(c, c % _NSLOTS, True, True)
            c += 1
        while c < per_worker:
            chunk_body(c, c % _NSLOTS, False, False)
            c += 1

        for s in range(_NSLOTS):
            wait_store(s)

    return gather_kernel(table, idx32)


def kernel(indices, table):
    batch, seq = indices.shape
    num_rows, dim = table.shape
    idx32 = indices.astype(jnp.int32)
    chunk_batches = batch // _K

    out = jnp.zeros((batch, seq, dim), table.dtype)
    for k in range(_K):
        chunk = _sc_gather_chunk(k, chunk_batches, seq, dim, table, idx32)
        out = lax.dynamic_update_slice(out, chunk, (k * chunk_batches, 0, 0))
    return out
